# Initial kernel scaffold; baseline (speedup 1.0000x reference)
import jax
import jax.numpy as jnp
from jax.experimental import pallas as pl


def kernel(vectors, node_feats, node_specie, senders, receivers, W_skip, W_up, M0, M1, M2, W_down, W_out):
    raise NotImplementedError("write your pallas kernel here")


# speedup vs baseline: 1.5352x; 1.5352x over previous
import functools

import jax
import jax.numpy as jnp
from jax import lax
from jax.experimental import pallas as pl
from jax.experimental.pallas import tpu as pltpu
from jax.experimental.pallas import tpu_sc as plsc

N_NODES = 10000
N_EDGES = 320000
D = 128
N_SPECIES = 5
N_BASIS = 8
MLP_HIDDEN = 64
CUTOFF = 2.0
AVG_NEIGHBORS = 32.0
SOFT_NORM = 100000.0


_BN = 1000


def _node_body(nf_ref, sp_ref, wskip_ref, wup_ref, h_ref, sc_ref):
    nf = nf_ref[...]
    h_ref[...] = jnp.dot(nf, wup_ref[...], preferred_element_type=jnp.float32)
    sp = sp_ref[...]
    acc = jnp.zeros((_BN, D), jnp.float32)
    for s in range(N_SPECIES):
        m = (sp == s).astype(jnp.float32)
        acc = acc + jnp.dot(nf * m, wskip_ref[s],
                            preferred_element_type=jnp.float32)
    sc_ref[...] = acc


def _node_dense(node_feats, specie2d, W_skip, W_up):
    grid = (N_NODES // _BN,)
    return pl.pallas_call(
        _node_body,
        grid=grid,
        in_specs=[
            pl.BlockSpec((_BN, D), lambda i: (i, 0)),
            pl.BlockSpec((_BN, 1), lambda i: (i, 0)),
            pl.BlockSpec((N_SPECIES, D, D), lambda i: (0, 0, 0)),
            pl.BlockSpec((D, D), lambda i: (0, 0)),
        ],
        out_specs=[
            pl.BlockSpec((_BN, D), lambda i: (i, 0)),
            pl.BlockSpec((_BN, D), lambda i: (i, 0)),
        ],
        out_shape=[
            jax.ShapeDtypeStruct((N_NODES, D), jnp.float32),
            jax.ShapeDtypeStruct((N_NODES, D), jnp.float32),
        ],
    )(node_feats, specie2d, W_skip, W_up)


_BE = 4000


def _mix_body(vec_ref, m0_ref, m1_ref, m2_ref, mix_ref):
    v = vec_ref[...]
    l2 = jnp.sum(v * v, axis=1, keepdims=True)
    x = jnp.sqrt(l2)
    ns = (lax.broadcasted_iota(jnp.int32, (1, N_BASIS), 1) + 1).astype(jnp.float32)
    c = CUTOFF
    x_safe = jnp.where(x == 0.0, 1.0, x)
    vals = jnp.sqrt(2.0 / c) * jnp.sin(ns * (jnp.pi / c) * x) / x_safe
    lim = jnp.sqrt(2.0 / c) * ns * (jnp.pi / c)
    rad = jnp.where(x == 0.0, lim, vals)
    t = jnp.clip(x / c, 0.0, 1.0)
    env = 1.0 - 10.0 * t ** 3 + 15.0 * t ** 4 - 6.0 * t ** 5
    rad = rad * env
    m = jax.nn.gelu(jnp.dot(rad, m0_ref[...], preferred_element_type=jnp.float32))
    m = jax.nn.gelu(jnp.dot(m, m1_ref[...], preferred_element_type=jnp.float32))
    mix = jnp.dot(m, m2_ref[...], preferred_element_type=jnp.float32)
    mix_ref[...] = jnp.where(x == 0.0, 0.0, mix)


def _edge_mix(vectors, M0, M1, M2):
    grid = (N_EDGES // _BE,)
    return pl.pallas_call(
        _mix_body,
        grid=grid,
        in_specs=[
            pl.BlockSpec((_BE, 3), lambda i: (i, 0)),
            pl.BlockSpec((N_BASIS, MLP_HIDDEN), lambda i: (0, 0)),
            pl.BlockSpec((MLP_HIDDEN, MLP_HIDDEN), lambda i: (0, 0)),
            pl.BlockSpec((MLP_HIDDEN, D), lambda i: (0, 0)),
        ],
        out_specs=pl.BlockSpec((_BE, D), lambda i: (i, 0)),
        out_shape=jax.ShapeDtypeStruct((N_EDGES, D), jnp.float32),
    )(vectors, M0, M1, M2)


_NW = 32
_NS = 16
_EPT = N_EDGES // _NW
_C = 80
_NCHUNK = _EPT // _C
_RPT = N_NODES // _NS


def _sc_body(h_hbm, mix_hbm, snd_hbm, rcv_hbm, zeros_hbm, out_hbm,
             snd_v, rcv_v, rows_v, mix_v, agg_sh, sem):
    c = lax.axis_index("c")
    s = lax.axis_index("s")
    wid = c * _NS + s

    @pl.when(s == 0)
    def _init():
        pltpu.sync_copy(zeros_hbm, agg_sh)

    plsc.subcore_barrier()

    base0 = wid * _EPT

    def chunk(k, carry):
        base = pl.multiple_of(base0 + k * _C, 8)
        pltpu.sync_copy(snd_hbm.at[pl.ds(base, _C)], snd_v)
        pltpu.sync_copy(rcv_hbm.at[pl.ds(base, _C)], rcv_v)
        gather = pltpu.async_copy(h_hbm.at[snd_v], rows_v, sem)
        pltpu.sync_copy(mix_hbm.at[pl.ds(base, _C)], mix_v)
        gather.wait()

        def mul_row(r, cy):
            for j in range(D // 16):
                sl = pl.ds(j * 16, 16)
                rows_v[r, sl] = rows_v[r, sl] * mix_v[r, sl]
            return cy

        lax.fori_loop(0, _C, mul_row, 0)
        pltpu.sync_copy(rows_v, agg_sh.at[rcv_v], add=True)
        return carry

    lax.fori_loop(0, _NCHUNK, chunk, 0)
    plsc.subcore_barrier()

    @pl.when(s == 0)
    def _publish():
        pltpu.sync_copy(agg_sh, out_hbm.at[pl.ds(c * N_NODES, N_NODES)])


def _sc_scatter(h, mix, senders, receivers, zeros):
    mesh = plsc.VectorSubcoreMesh(core_axis_name="c", subcore_axis_name="s")
    kern = functools.partial(
        pl.kernel,
        mesh=mesh,
        out_type=jax.ShapeDtypeStruct((2 * N_NODES, D), jnp.float32),
        scratch_types=[
            pltpu.VMEM((_C,), jnp.int32),
            pltpu.VMEM((_C,), jnp.int32),
            pltpu.VMEM((_C, D), jnp.float32),
            pltpu.VMEM((_C, D), jnp.float32),
            pltpu.VMEM_SHARED((N_NODES, D), jnp.float32),
            pltpu.SemaphoreType.DMA,
        ],
    )(_sc_body)
    return kern(h, mix, senders, receivers, zeros)


def _post_body(p_ref, scon_ref, wd_ref, wo_ref, out_ref):
    p = p_ref[...]
    agg = (p[0] + p[1]) * (1.0 / jnp.sqrt(AVG_NEIGHBORS))
    t = jax.nn.gelu(jnp.dot(agg, wd_ref[...], preferred_element_type=jnp.float32))
    t = jnp.dot(t, wo_ref[...], preferred_element_type=jnp.float32)
    n = jnp.sqrt(jnp.sum(t * t, axis=1, keepdims=True)) / SOFT_NORM
    n_safe = jnp.where(n > 0.0, n, 1.0)
    sus = jnp.where(n > 0.0, jnp.exp(-1.0 / n_safe), 0.0)
    phi = 1.0 / (1.0 + n * sus)
    out_ref[...] = 0.9 * scon_ref[...] + 0.45 * (t * phi)


def _post(parts, scon, W_down, W_out):
    grid = (N_NODES // _BN,)
    return pl.pallas_call(
        _post_body,
        grid=grid,
        in_specs=[
            pl.BlockSpec((2, _BN, D), lambda i: (0, i, 0)),
            pl.BlockSpec((_BN, D), lambda i: (i, 0)),
            pl.BlockSpec((D, D), lambda i: (0, 0)),
            pl.BlockSpec((D, D), lambda i: (0, 0)),
        ],
        out_specs=pl.BlockSpec((_BN, D), lambda i: (i, 0)),
        out_shape=jax.ShapeDtypeStruct((N_NODES, D), jnp.float32),
    )(parts, scon, W_down, W_out)


def kernel(vectors, node_feats, node_specie, senders, receivers,
           W_skip, W_up, M0, M1, M2, W_down, W_out):
    specie2d = node_specie.astype(jnp.int32).reshape(N_NODES, 1)
    h, scon = _node_dense(node_feats, specie2d, W_skip, W_up)
    mix = _edge_mix(vectors, M0, M1, M2)
    zeros = jnp.zeros((N_NODES, D), jnp.float32)
    parts = _sc_scatter(h, mix, senders.astype(jnp.int32),
                        receivers.astype(jnp.int32), zeros)
    return _post(parts.reshape(2, N_NODES, D), scon, W_down, W_out)


# speedup vs baseline: 2.4230x; 1.5783x over previous
import functools

import jax
import jax.numpy as jnp
from jax import lax
from jax.experimental import pallas as pl
from jax.experimental.pallas import tpu as pltpu
from jax.experimental.pallas import tpu_sc as plsc

N_NODES = 10000
N_EDGES = 320000
D = 128
N_SPECIES = 5
N_BASIS = 8
MLP_HIDDEN = 64
CUTOFF = 2.0
AVG_NEIGHBORS = 32.0
SOFT_NORM = 100000.0


_BN = 1000


def _node_body(nf_ref, sp_ref, wskip_ref, wup_ref, h_ref, sc_ref):
    nf = nf_ref[...]
    h_ref[...] = jnp.dot(nf, wup_ref[...], preferred_element_type=jnp.float32)
    sp = sp_ref[...]
    acc = jnp.zeros((_BN, D), jnp.float32)
    for s in range(N_SPECIES):
        m = (sp == s).astype(jnp.float32)
        acc = acc + jnp.dot(nf * m, wskip_ref[s],
                            preferred_element_type=jnp.float32)
    sc_ref[...] = acc


def _node_dense(node_feats, specie2d, W_skip, W_up):
    grid = (N_NODES // _BN,)
    return pl.pallas_call(
        _node_body,
        grid=grid,
        in_specs=[
            pl.BlockSpec((_BN, D), lambda i: (i, 0)),
            pl.BlockSpec((_BN, 1), lambda i: (i, 0)),
            pl.BlockSpec((N_SPECIES, D, D), lambda i: (0, 0, 0)),
            pl.BlockSpec((D, D), lambda i: (0, 0)),
        ],
        out_specs=[
            pl.BlockSpec((_BN, D), lambda i: (i, 0)),
            pl.BlockSpec((_BN, D), lambda i: (i, 0)),
        ],
        out_shape=[
            jax.ShapeDtypeStruct((N_NODES, D), jnp.float32),
            jax.ShapeDtypeStruct((N_NODES, D), jnp.float32),
        ],
    )(node_feats, specie2d, W_skip, W_up)


_BE = 4000


def _mix_body(vec_ref, m0_ref, m1_ref, m2_ref, mix_ref):
    v = vec_ref[...]
    l2 = jnp.sum(v * v, axis=1, keepdims=True)
    x = jnp.sqrt(l2)
    ns = (lax.broadcasted_iota(jnp.int32, (1, N_BASIS), 1) + 1).astype(jnp.float32)
    c = CUTOFF
    x_safe = jnp.where(x == 0.0, 1.0, x)
    u = (jnp.pi / c) * jnp.minimum(x, c) - (0.5 * jnp.pi)
    t2 = u * u
    s1 = 1.0 + t2 * (-1.0 / 2 + t2 * (1.0 / 24 + t2 * (-1.0 / 720
         + t2 * (1.0 / 40320 + t2 * (-1.0 / 3628800)))))
    c1 = -u * (1.0 + t2 * (-1.0 / 6 + t2 * (1.0 / 120 + t2 * (-1.0 / 5040
         + t2 * (1.0 / 362880 + t2 * (-1.0 / 39916800))))))
    two_c1 = 2.0 * c1
    s_prev = jnp.zeros_like(s1)
    s_cur = s1
    cols = []
    for _ in range(N_BASIS):
        cols.append(s_cur)
        s_prev, s_cur = s_cur, two_c1 * s_cur - s_prev
    sins = jnp.concatenate(cols, axis=1)
    vals = jnp.sqrt(2.0 / c) * sins / x_safe
    lim = jnp.sqrt(2.0 / c) * ns * (jnp.pi / c)
    rad = jnp.where(x == 0.0, lim, vals)
    t = jnp.clip(x / c, 0.0, 1.0)
    env = 1.0 - 10.0 * t ** 3 + 15.0 * t ** 4 - 6.0 * t ** 5
    rad = rad * env
    m = jax.nn.gelu(jnp.dot(rad, m0_ref[...], preferred_element_type=jnp.float32))
    m = jax.nn.gelu(jnp.dot(m, m1_ref[...], preferred_element_type=jnp.float32))
    mix = jnp.dot(m, m2_ref[...], preferred_element_type=jnp.float32)
    mix_ref[...] = jnp.where(x == 0.0, 0.0, mix)


def _edge_mix(vectors, M0, M1, M2):
    grid = (N_EDGES // _BE,)
    return pl.pallas_call(
        _mix_body,
        grid=grid,
        in_specs=[
            pl.BlockSpec((_BE, 3), lambda i: (i, 0)),
            pl.BlockSpec((N_BASIS, MLP_HIDDEN), lambda i: (0, 0)),
            pl.BlockSpec((MLP_HIDDEN, MLP_HIDDEN), lambda i: (0, 0)),
            pl.BlockSpec((MLP_HIDDEN, D), lambda i: (0, 0)),
        ],
        out_specs=pl.BlockSpec((_BE, D), lambda i: (i, 0)),
        out_shape=jax.ShapeDtypeStruct((N_EDGES, D), jnp.float32),
    )(vectors, M0, M1, M2)


_NW = 32
_NS = 16
_EPT = N_EDGES // _NW
_C = 40
_NCHUNK = _EPT // _C


def _sc_body(h_hbm, mix_hbm, snd_hbm, rcv_hbm, zeros_hbm, out_hbm,
             snd_all, rcv_all, rcv_v, rows, mixb, agg_sh,
             sem_g0, sem_g1, sem_m0, sem_m1):
    c = lax.axis_index("c")
    s = lax.axis_index("s")
    wid = c * _NS + s

    @pl.when(s == 0)
    def _init():
        pltpu.sync_copy(zeros_hbm, agg_sh)

    ebase = wid * _EPT
    pltpu.sync_copy(snd_hbm.at[pl.ds(ebase, _EPT)], snd_all)
    pltpu.sync_copy(rcv_hbm.at[pl.ds(ebase, _EPT)], rcv_all)
    plsc.subcore_barrier()

    sem_g = (sem_g0, sem_g1)
    sem_m = (sem_m0, sem_m1)

    def start(k, b):
        off = pl.multiple_of(k * _C, 8)
        pltpu.async_copy(h_hbm.at[snd_all.at[pl.ds(off, _C)]],
                         rows.at[b], sem_g[b])
        hoff = pl.multiple_of(ebase + k * _C, 8)
        pltpu.async_copy(mix_hbm.at[pl.ds(hoff, _C)], mixb.at[b], sem_m[b])

    def compute(k, b):
        off = k * _C
        for j0 in (0, 16, _C - 16):
            sl = pl.ds(j0, 16)
            rcv_v[b, sl] = rcv_all[pl.ds(pl.multiple_of(off + j0, 8), 16)]
        pltpu.make_async_copy(h_hbm.at[pl.ds(0, _C)], rows.at[b],
                              sem_g[b]).wait()
        pltpu.make_async_copy(mix_hbm.at[pl.ds(0, _C)], mixb.at[b],
                              sem_m[b]).wait()

        def mul_row(r, cy):
            for j in range(D // 16):
                sl = pl.ds(j * 16, 16)
                rows[b, r, sl] = rows[b, r, sl] * mixb[b, r, sl]
            return cy

        lax.fori_loop(0, _C, mul_row, 0)
        pltpu.sync_copy(rows.at[b], agg_sh.at[rcv_v.at[b]], add=True)

    start(0, 0)
    start(1, 1)

    def pair(i, cy):
        k0 = 2 * i

        compute(k0, 0)

        @pl.when(k0 + 2 < _NCHUNK)
        def _s0():
            start(k0 + 2, 0)

        compute(k0 + 1, 1)

        @pl.when(k0 + 3 < _NCHUNK)
        def _s1():
            start(k0 + 3, 1)

        return cy

    lax.fori_loop(0, _NCHUNK // 2, pair, 0)
    plsc.subcore_barrier()

    @pl.when(s == 0)
    def _publish():
        pltpu.sync_copy(agg_sh, out_hbm.at[pl.ds(c * N_NODES, N_NODES)])


def _sc_scatter(h, mix, senders, receivers, zeros):
    mesh = plsc.VectorSubcoreMesh(core_axis_name="c", subcore_axis_name="s")
    kern = functools.partial(
        pl.kernel,
        mesh=mesh,
        out_type=jax.ShapeDtypeStruct((2 * N_NODES, D), jnp.float32),
        scratch_types=[
            pltpu.VMEM((_EPT,), jnp.int32),
            pltpu.VMEM((_EPT,), jnp.int32),
            pltpu.VMEM((2, _C), jnp.int32),
            pltpu.VMEM((2, _C, D), jnp.float32),
            pltpu.VMEM((2, _C, D), jnp.float32),
            pltpu.VMEM_SHARED((N_NODES, D), jnp.float32),
            pltpu.SemaphoreType.DMA,
            pltpu.SemaphoreType.DMA,
            pltpu.SemaphoreType.DMA,
            pltpu.SemaphoreType.DMA,
        ],
    )(_sc_body)
    return kern(h, mix, senders, receivers, zeros)


def _post_body(p_ref, scon_ref, wd_ref, wo_ref, out_ref):
    p = p_ref[...]
    agg = (p[0] + p[1]) * (1.0 / jnp.sqrt(AVG_NEIGHBORS))
    t = jax.nn.gelu(jnp.dot(agg, wd_ref[...], preferred_element_type=jnp.float32))
    t = jnp.dot(t, wo_ref[...], preferred_element_type=jnp.float32)
    n = jnp.sqrt(jnp.sum(t * t, axis=1, keepdims=True)) / SOFT_NORM
    n_safe = jnp.where(n > 0.0, n, 1.0)
    sus = jnp.where(n > 0.0, jnp.exp(-1.0 / n_safe), 0.0)
    phi = 1.0 / (1.0 + n * sus)
    out_ref[...] = 0.9 * scon_ref[...] + 0.45 * (t * phi)


def _post(parts, scon, W_down, W_out):
    grid = (N_NODES // _BN,)
    return pl.pallas_call(
        _post_body,
        grid=grid,
        in_specs=[
            pl.BlockSpec((2, _BN, D), lambda i: (0, i, 0)),
            pl.BlockSpec((_BN, D), lambda i: (i, 0)),
            pl.BlockSpec((D, D), lambda i: (0, 0)),
            pl.BlockSpec((D, D), lambda i: (0, 0)),
        ],
        out_specs=pl.BlockSpec((_BN, D), lambda i: (i, 0)),
        out_shape=jax.ShapeDtypeStruct((N_NODES, D), jnp.float32),
    )(parts, scon, W_down, W_out)


def kernel(vectors, node_feats, node_specie, senders, receivers,
           W_skip, W_up, M0, M1, M2, W_down, W_out):
    specie2d = node_specie.astype(jnp.int32).reshape(N_NODES, 1)
    h, scon = _node_dense(node_feats, specie2d, W_skip, W_up)
    mix = _edge_mix(vectors, M0, M1, M2)
    zeros = jnp.zeros((N_NODES, D), jnp.float32)
    parts = _sc_scatter(h, mix, senders.astype(jnp.int32),
                        receivers.astype(jnp.int32), zeros)
    return _post(parts.reshape(2, N_NODES, D), scon, W_down, W_out)


# speedup vs baseline: 5.0797x; 2.0965x over previous
import functools

import jax
import jax.numpy as jnp
from jax import lax
from jax.experimental import pallas as pl
from jax.experimental.pallas import tpu as pltpu
from jax.experimental.pallas import tpu_sc as plsc

N_NODES = 10000
N_EDGES = 320000
D = 128
N_SPECIES = 5
N_BASIS = 8
MLP_HIDDEN = 64
CUTOFF = 2.0
AVG_NEIGHBORS = 32.0
SOFT_NORM = 100000.0


_BN = 1000


def _node_body(nf_ref, sp_ref, wskip_ref, wup_ref, h_ref, sc_ref):
    nf = nf_ref[...]
    h_ref[...] = jnp.dot(nf, wup_ref[...], preferred_element_type=jnp.float32)
    sp = sp_ref[...]
    acc = jnp.zeros((_BN, D), jnp.float32)
    for s in range(N_SPECIES):
        m = (sp == s).astype(jnp.float32)
        acc = acc + jnp.dot(nf * m, wskip_ref[s],
                            preferred_element_type=jnp.float32)
    sc_ref[...] = acc


def _node_dense(node_feats, specie2d, W_skip, W_up):
    grid = (N_NODES // _BN,)
    return pl.pallas_call(
        _node_body,
        grid=grid,
        in_specs=[
            pl.BlockSpec((_BN, D), lambda i: (i, 0)),
            pl.BlockSpec((_BN, 1), lambda i: (i, 0)),
            pl.BlockSpec((N_SPECIES, D, D), lambda i: (0, 0, 0)),
            pl.BlockSpec((D, D), lambda i: (0, 0)),
        ],
        out_specs=[
            pl.BlockSpec((_BN, D), lambda i: (i, 0)),
            pl.BlockSpec((_BN, D), lambda i: (i, 0)),
        ],
        out_shape=[
            jax.ShapeDtypeStruct((N_NODES, D), jnp.float32),
            jax.ShapeDtypeStruct((N_NODES, D), jnp.float32),
        ],
    )(node_feats, specie2d, W_skip, W_up)


_BE = 3200


def _mix_body(vt_ref, m0t_ref, m1t_ref, m2t_ref, mix_ref):
    v0 = vt_ref[0:1, :]
    v1 = vt_ref[1:2, :]
    v2 = vt_ref[2:3, :]
    l2 = v0 * v0 + v1 * v1 + v2 * v2
    x = jnp.sqrt(l2)
    c = CUTOFF
    xz = x == 0.0
    x_safe = jnp.where(xz, 1.0, x)
    u = (jnp.pi / c) * jnp.minimum(x, c) - (0.5 * jnp.pi)
    t2 = u * u
    s1 = 1.0 + t2 * (-1.0 / 2 + t2 * (1.0 / 24 + t2 * (-1.0 / 720
         + t2 * (1.0 / 40320 + t2 * (-1.0 / 3628800)))))
    c1 = -u * (1.0 + t2 * (-1.0 / 6 + t2 * (1.0 / 120 + t2 * (-1.0 / 5040
         + t2 * (1.0 / 362880 + t2 * (-1.0 / 39916800))))))
    tt = jnp.clip(x / c, 0.0, 1.0)
    env = 1.0 - 10.0 * tt ** 3 + 15.0 * tt ** 4 - 6.0 * tt ** 5
    sq2c = jnp.sqrt(2.0 / c)
    kk = sq2c * env / x_safe
    two_c1 = 2.0 * c1
    s_prev = jnp.zeros_like(s1)
    s_cur = s1
    cols = []
    for n in range(1, N_BASIS + 1):
        lim_n = sq2c * n * jnp.pi / c
        cols.append(jnp.where(xz, lim_n, s_cur * kk))
        s_prev, s_cur = s_cur, two_c1 * s_cur - s_prev
    radt = jnp.concatenate(cols, axis=0)
    mt = jax.nn.gelu(jnp.dot(m0t_ref[...], radt,
                             preferred_element_type=jnp.float32))
    mt = jax.nn.gelu(jnp.dot(m1t_ref[...], mt,
                             preferred_element_type=jnp.float32))
    mixt = jnp.dot(m2t_ref[...], mt, preferred_element_type=jnp.float32)
    mixt = jnp.where(xz, 0.0, mixt)
    ii = lax.broadcasted_iota(jnp.int32, (D, D), 0)
    jj = lax.broadcasted_iota(jnp.int32, (D, D), 1)
    eye = (ii == jj).astype(jnp.float32)
    mix_ref[...] = lax.dot_general(mixt, eye, (((0,), (0,)), ((), ())),
                                   preferred_element_type=jnp.float32)


def _edge_mix(vectors_t, M0t, M1t, M2t):
    grid = (N_EDGES // _BE,)
    return pl.pallas_call(
        _mix_body,
        grid=grid,
        in_specs=[
            pl.BlockSpec((3, _BE), lambda i: (0, i)),
            pl.BlockSpec((MLP_HIDDEN, N_BASIS), lambda i: (0, 0)),
            pl.BlockSpec((MLP_HIDDEN, MLP_HIDDEN), lambda i: (0, 0)),
            pl.BlockSpec((D, MLP_HIDDEN), lambda i: (0, 0)),
        ],
        out_specs=pl.BlockSpec((_BE, D), lambda i: (i, 0)),
        out_shape=jax.ShapeDtypeStruct((N_EDGES, D), jnp.float32),
    )(vectors_t, M0t, M1t, M2t)


_NW = 32
_NS = 16
_EPT = N_EDGES // _NW
_C = 40
_NCHUNK = _EPT // _C


def _sc_body(h_hbm, mix_hbm, snd_hbm, rcv_hbm, zeros_hbm, out_hbm,
             snd_all, rcv_all, rcv_v, rows, mixb, agg_sh,
             sem_g0, sem_g1, sem_m0, sem_m1):
    c = lax.axis_index("c")
    s = lax.axis_index("s")
    wid = c * _NS + s

    @pl.when(s == 0)
    def _init():
        pltpu.sync_copy(zeros_hbm, agg_sh)

    ebase = wid * _EPT
    pltpu.sync_copy(snd_hbm.at[pl.ds(ebase, _EPT)], snd_all)
    pltpu.sync_copy(rcv_hbm.at[pl.ds(ebase, _EPT)], rcv_all)
    plsc.subcore_barrier()

    sem_g = (sem_g0, sem_g1)
    sem_m = (sem_m0, sem_m1)

    def start(k, b):
        off = pl.multiple_of(k * _C, 8)
        pltpu.async_copy(h_hbm.at[snd_all.at[pl.ds(off, _C)]],
                         rows.at[b], sem_g[b])
        hoff = pl.multiple_of(ebase + k * _C, 8)
        pltpu.async_copy(mix_hbm.at[pl.ds(hoff, _C)], mixb.at[b], sem_m[b])

    def compute(k, b):
        off = k * _C
        for j0 in (0, 16, _C - 16):
            sl = pl.ds(j0, 16)
            rcv_v[b, sl] = rcv_all[pl.ds(pl.multiple_of(off + j0, 8), 16)]
        pltpu.make_async_copy(h_hbm.at[pl.ds(0, _C)], rows.at[b],
                              sem_g[b]).wait()
        pltpu.make_async_copy(mix_hbm.at[pl.ds(0, _C)], mixb.at[b],
                              sem_m[b]).wait()

        def mul_row(r, cy):
            for j in range(D // 16):
                sl = pl.ds(j * 16, 16)
                rows[b, r, sl] = rows[b, r, sl] * mixb[b, r, sl]
            return cy

        lax.fori_loop(0, _C, mul_row, 0)
        pltpu.sync_copy(rows.at[b], agg_sh.at[rcv_v.at[b]], add=True)

    start(0, 0)
    start(1, 1)

    def pair(i, cy):
        k0 = 2 * i

        compute(k0, 0)

        @pl.when(k0 + 2 < _NCHUNK)
        def _s0():
            start(k0 + 2, 0)

        compute(k0 + 1, 1)

        @pl.when(k0 + 3 < _NCHUNK)
        def _s1():
            start(k0 + 3, 1)

        return cy

    lax.fori_loop(0, _NCHUNK // 2, pair, 0)
    plsc.subcore_barrier()

    @pl.when(s == 0)
    def _publish():
        pltpu.sync_copy(agg_sh, out_hbm.at[pl.ds(c * N_NODES, N_NODES)])


def _sc_scatter(h, mix, senders, receivers, zeros):
    mesh = plsc.VectorSubcoreMesh(core_axis_name="c", subcore_axis_name="s")
    kern = functools.partial(
        pl.kernel,
        mesh=mesh,
        out_type=jax.ShapeDtypeStruct((2 * N_NODES, D), jnp.float32),
        scratch_types=[
            pltpu.VMEM((_EPT,), jnp.int32),
            pltpu.VMEM((_EPT,), jnp.int32),
            pltpu.VMEM((2, _C), jnp.int32),
            pltpu.VMEM((2, _C, D), jnp.float32),
            pltpu.VMEM((2, _C, D), jnp.float32),
            pltpu.VMEM_SHARED((N_NODES, D), jnp.float32),
            pltpu.SemaphoreType.DMA,
            pltpu.SemaphoreType.DMA,
            pltpu.SemaphoreType.DMA,
            pltpu.SemaphoreType.DMA,
        ],
    )(_sc_body)
    return kern(h, mix, senders, receivers, zeros)


def _post_body(p_ref, scon_ref, wd_ref, wo_ref, out_ref):
    p = p_ref[...]
    agg = (p[0] + p[1]) * (1.0 / jnp.sqrt(AVG_NEIGHBORS))
    t = jax.nn.gelu(jnp.dot(agg, wd_ref[...], preferred_element_type=jnp.float32))
    t = jnp.dot(t, wo_ref[...], preferred_element_type=jnp.float32)
    n = jnp.sqrt(jnp.sum(t * t, axis=1, keepdims=True)) / SOFT_NORM
    n_safe = jnp.where(n > 0.0, n, 1.0)
    sus = jnp.where(n > 0.0, jnp.exp(-1.0 / n_safe), 0.0)
    phi = 1.0 / (1.0 + n * sus)
    out_ref[...] = 0.9 * scon_ref[...] + 0.45 * (t * phi)


def _post(parts, scon, W_down, W_out):
    grid = (N_NODES // _BN,)
    return pl.pallas_call(
        _post_body,
        grid=grid,
        in_specs=[
            pl.BlockSpec((2, _BN, D), lambda i: (0, i, 0)),
            pl.BlockSpec((_BN, D), lambda i: (i, 0)),
            pl.BlockSpec((D, D), lambda i: (0, 0)),
            pl.BlockSpec((D, D), lambda i: (0, 0)),
        ],
        out_specs=pl.BlockSpec((_BN, D), lambda i: (i, 0)),
        out_shape=jax.ShapeDtypeStruct((N_NODES, D), jnp.float32),
    )(parts, scon, W_down, W_out)


def kernel(vectors, node_feats, node_specie, senders, receivers,
           W_skip, W_up, M0, M1, M2, W_down, W_out):
    specie2d = node_specie.astype(jnp.int32).reshape(N_NODES, 1)
    h, scon = _node_dense(node_feats, specie2d, W_skip, W_up)
    mix = _edge_mix(vectors.T, M0.T, M1.T, M2.T)
    zeros = jnp.zeros((N_NODES, D), jnp.float32)
    parts = _sc_scatter(h, mix, senders.astype(jnp.int32),
                        receivers.astype(jnp.int32), zeros)
    return _post(parts.reshape(2, N_NODES, D), scon, W_down, W_out)
